# baseline (speedup 1.0000x reference)
import functools

import jax
import jax.numpy as jnp
from jax import lax
from jax.experimental import pallas as pl
from jax.experimental.pallas import tpu as pltpu
from jax.experimental.pallas import tpu_sc as plsc

RAND_NEG_RATIO = 3
LEAST_NEG_PERCENT = 0.05
LOSS_WEIGHT = 1.0

_NW = 32
_VEC = 16
_CHUNK = 16384
_UNROLL = 8

_Q = (0.999991828530996, -0.49937259784652266, 0.3252951414015596,
      -0.21029369270422338, 0.1015000471540588, -0.023979573072245318)


def _log1p_poly(u):
    q = jnp.full_like(u, _Q[5])
    for c in (_Q[4], _Q[3], _Q[2], _Q[1], _Q[0]):
        q = q * u + c
    return u * q


def _sc_partials(pred_flat, label_flat):
    total = pred_flat.shape[0]
    per_w = total // _NW
    nchunk = per_w // _CHUNK
    mesh = plsc.VectorSubcoreMesh(core_axis_name="c", subcore_axis_name="s")

    @functools.partial(
        pl.kernel,
        mesh=mesh,
        out_type=(jax.ShapeDtypeStruct((_NW, _VEC), jnp.float32),
                  jax.ShapeDtypeStruct((_NW, _VEC), jnp.float32)),
        scratch_types=[
            pltpu.VMEM((2, _CHUNK), jnp.float32),
            pltpu.VMEM((2, _CHUNK), jnp.float32),
            pltpu.VMEM((_VEC,), jnp.float32),
            pltpu.VMEM((_VEC,), jnp.float32),
            pltpu.SemaphoreType.DMA,
            pltpu.SemaphoreType.DMA,
        ],
    )
    def k(p_hbm, l_hbm, s_out, n_out, pbuf, lbuf, svec, nvec, psem, lsem):
        cid = lax.axis_index("c")
        sid = lax.axis_index("s")
        wid = sid * 2 + cid
        base = wid * per_w

        hp = [None, None]
        hl = [None, None]
        hp[0] = pltpu.async_copy(p_hbm.at[pl.ds(base, _CHUNK)], pbuf.at[0], psem)
        hl[0] = pltpu.async_copy(l_hbm.at[pl.ds(base, _CHUNK)], lbuf.at[0], lsem)

        acc = (jnp.zeros((_VEC,), jnp.float32),
               jnp.zeros((_VEC,), jnp.float32),
               jnp.zeros((_VEC,), jnp.float32))
        for c in range(nchunk):
            cur, nxt = c % 2, (c + 1) % 2
            if c + 1 < nchunk:
                off = base + (c + 1) * _CHUNK
                hp[nxt] = pltpu.async_copy(p_hbm.at[pl.ds(off, _CHUNK)],
                                           pbuf.at[nxt], psem)
                hl[nxt] = pltpu.async_copy(l_hbm.at[pl.ds(off, _CHUNK)],
                                           lbuf.at[nxt], lsem)
            hp[cur].wait()
            hl[cur].wait()

            def body(j, carry, cur=cur):
                am, aw, al = carry
                off = j * (_VEC * _UNROLL)
                for k2 in range(_UNROLL):
                    p = pbuf[cur, pl.ds(off + k2 * _VEC, _VEC)]
                    l = lbuf[cur, pl.ds(off + k2 * _VEC, _VEC)]
                    u = jnp.exp(-jnp.abs(p))
                    am = am + (jnp.maximum(p, 0.0) - p * l)
                    aw = aw + _log1p_poly(u)
                    al = al + l
                return am, aw, al

            acc = lax.fori_loop(0, _CHUNK // (_VEC * _UNROLL), body, acc)

        svec[...] = acc[0] + acc[1]
        nvec[...] = acc[2]
        pltpu.sync_copy(svec, s_out.at[wid])
        pltpu.sync_copy(nvec, n_out.at[wid])

    return k(pred_flat, label_flat)


def kernel(pred, label):
    total = pred.size
    s_parts, n_parts = _sc_partials(pred.reshape(-1), label.reshape(-1))
    num_pos = jnp.sum(n_parts)
    least = float(int(total * LEAST_NEG_PERCENT))
    rand_neg = jnp.maximum(num_pos * float(RAND_NEG_RATIO), least)
    num_sampled_neg = jnp.minimum(rand_neg, float(total) - num_pos)
    balanced = num_pos + num_sampled_neg
    return LOSS_WEIGHT * jnp.sum(s_parts) / balanced


# speedup vs baseline: 1.4897x; 1.4897x over previous
import functools

import jax
import jax.numpy as jnp
from jax import lax
from jax.experimental import pallas as pl
from jax.experimental.pallas import tpu as pltpu
from jax.experimental.pallas import tpu_sc as plsc

RAND_NEG_RATIO = 3
LEAST_NEG_PERCENT = 0.05
LOSS_WEIGHT = 1.0

_LANES = 512
_NBLK = 4
_STRIPE = 8
_TC_FRAC_NUM, _TC_FRAC_DEN = 7, 8

_NW = 32
_VEC = 16
_CHUNK = 16384
_UNROLL = 8

_Q = (0.999991828530996, -0.49937259784652266, 0.3252951414015596,
      -0.21029369270422338, 0.1015000471540588, -0.023979573072245318)


def _log1p_poly(u):
    q = jnp.full_like(u, _Q[5])
    for c in (_Q[4], _Q[3], _Q[2], _Q[1], _Q[0]):
        q = q * u + c
    return u * q


def _tc_body(p_ref, l_ref, out_ref, macc_ref, wacc_ref, lacc_ref, *, nblk):
    i = pl.program_id(0)

    @pl.when(i == 0)
    def _init():
        macc_ref[...] = jnp.zeros_like(macc_ref)
        wacc_ref[...] = jnp.zeros_like(wacc_ref)
        lacc_ref[...] = jnp.zeros_like(lacc_ref)

    blk_rows = p_ref.shape[0]
    z = jnp.zeros((_STRIPE, _LANES), jnp.float32)
    macc, wacc, lacc = z, z, z
    for k in range(blk_rows // _STRIPE):
        p = p_ref[k * _STRIPE:(k + 1) * _STRIPE, :]
        l = l_ref[k * _STRIPE:(k + 1) * _STRIPE, :]
        u = jnp.exp(-jnp.abs(p))
        wacc = wacc + jnp.log(u + 1.0)
        macc = macc + (jnp.maximum(p, 0.0) - p * l)
        lacc = lacc + l
    macc_ref[...] += macc
    wacc_ref[...] += wacc
    lacc_ref[...] += lacc

    @pl.when(i == nblk - 1)
    def _fin():
        out_ref[0] = jnp.sum(macc_ref[...]) + jnp.sum(wacc_ref[...])
        out_ref[1] = jnp.sum(lacc_ref[...])


def _tc_partials(p2, l2, tc_rows):
    blk_rows = tc_rows // _NBLK
    return pl.pallas_call(
        functools.partial(_tc_body, nblk=_NBLK),
        grid=(_NBLK,),
        in_specs=[
            pl.BlockSpec((blk_rows, _LANES), lambda i: (i, 0)),
            pl.BlockSpec((blk_rows, _LANES), lambda i: (i, 0)),
        ],
        out_specs=pl.BlockSpec(memory_space=pltpu.SMEM),
        out_shape=jax.ShapeDtypeStruct((2,), jnp.float32),
        scratch_shapes=[
            pltpu.VMEM((_STRIPE, _LANES), jnp.float32),
            pltpu.VMEM((_STRIPE, _LANES), jnp.float32),
            pltpu.VMEM((_STRIPE, _LANES), jnp.float32),
        ],
    )(p2, l2)


def _sc_partials(pred_flat, label_flat, sc_base, sc_elems):
    per_w = sc_elems // _NW
    nchunk = max(per_w // _CHUNK, 1)
    chunk = min(per_w, _CHUNK)
    mesh = plsc.VectorSubcoreMesh(core_axis_name="c", subcore_axis_name="s")

    @functools.partial(
        pl.kernel,
        mesh=mesh,
        out_type=(jax.ShapeDtypeStruct((_NW, _VEC), jnp.float32),
                  jax.ShapeDtypeStruct((_NW, _VEC), jnp.float32)),
        scratch_types=[
            pltpu.VMEM((2, chunk), jnp.float32),
            pltpu.VMEM((2, chunk), jnp.float32),
            pltpu.VMEM((_VEC,), jnp.float32),
            pltpu.VMEM((_VEC,), jnp.float32),
            pltpu.SemaphoreType.DMA,
            pltpu.SemaphoreType.DMA,
        ],
    )
    def k(p_hbm, l_hbm, s_out, n_out, pbuf, lbuf, svec, nvec, psem, lsem):
        cid = lax.axis_index("c")
        sid = lax.axis_index("s")
        wid = sid * 2 + cid
        base = sc_base + wid * per_w

        hp = [None, None]
        hl = [None, None]
        hp[0] = pltpu.async_copy(p_hbm.at[pl.ds(base, chunk)], pbuf.at[0], psem)
        hl[0] = pltpu.async_copy(l_hbm.at[pl.ds(base, chunk)], lbuf.at[0], lsem)

        acc = (jnp.zeros((_VEC,), jnp.float32),
               jnp.zeros((_VEC,), jnp.float32),
               jnp.zeros((_VEC,), jnp.float32))
        for c in range(nchunk):
            cur, nxt = c % 2, (c + 1) % 2
            if c + 1 < nchunk:
                off = base + (c + 1) * chunk
                hp[nxt] = pltpu.async_copy(p_hbm.at[pl.ds(off, chunk)],
                                           pbuf.at[nxt], psem)
                hl[nxt] = pltpu.async_copy(l_hbm.at[pl.ds(off, chunk)],
                                           lbuf.at[nxt], lsem)
            hp[cur].wait()
            hl[cur].wait()

            def body(j, carry, cur=cur):
                am, aw, al = carry
                off = j * (_VEC * _UNROLL)
                for k2 in range(_UNROLL):
                    p = pbuf[cur, pl.ds(off + k2 * _VEC, _VEC)]
                    l = lbuf[cur, pl.ds(off + k2 * _VEC, _VEC)]
                    u = jnp.exp(-jnp.abs(p))
                    am = am + (jnp.maximum(p, 0.0) - p * l)
                    aw = aw + _log1p_poly(u)
                    al = al + l
                return am, aw, al

            acc = lax.fori_loop(0, chunk // (_VEC * _UNROLL), body, acc)

        svec[...] = acc[0] + acc[1]
        nvec[...] = acc[2]
        pltpu.sync_copy(svec, s_out.at[wid])
        pltpu.sync_copy(nvec, n_out.at[wid])

    return k(pred_flat, label_flat)


def kernel(pred, label):
    total = pred.size
    rows = total // _LANES
    tc_rows = rows * _TC_FRAC_NUM // _TC_FRAC_DEN
    sc_base = tc_rows * _LANES
    sc_elems = total - sc_base

    p2 = pred.reshape(rows, _LANES)
    l2 = label.reshape(rows, _LANES)
    tc_out = _tc_partials(p2, l2, tc_rows)
    s_parts, n_parts = _sc_partials(pred.reshape(-1), label.reshape(-1),
                                    sc_base, sc_elems)

    s = tc_out[0] + jnp.sum(s_parts)
    num_pos = tc_out[1] + jnp.sum(n_parts)
    least = float(int(total * LEAST_NEG_PERCENT))
    rand_neg = jnp.maximum(num_pos * float(RAND_NEG_RATIO), least)
    num_sampled_neg = jnp.minimum(rand_neg, float(total) - num_pos)
    balanced = num_pos + num_sampled_neg
    return LOSS_WEIGHT * s / balanced


# speedup vs baseline: 6.9457x; 4.6625x over previous
import functools

import jax
import jax.numpy as jnp
from jax.experimental import pallas as pl
from jax.experimental.pallas import tpu as pltpu

RAND_NEG_RATIO = 3
LEAST_NEG_PERCENT = 0.05
LOSS_WEIGHT = 1.0

_LANES = 512
_NBLK = 4
_STRIPE = 16
_LOG2E = 1.4426950408889634
_LN2 = 0.6931471805599453


def _body(p_ref, l_ref, out_ref, macc_ref, wacc_ref, lacc_ref, *, nblk, total):
    i = pl.program_id(0)

    @pl.when(i == 0)
    def _init():
        macc_ref[...] = jnp.zeros_like(macc_ref)
        wacc_ref[...] = jnp.zeros_like(wacc_ref)
        lacc_ref[...] = jnp.zeros_like(lacc_ref)

    blk_rows = p_ref.shape[0]
    z = jnp.zeros((_STRIPE, _LANES), jnp.float32)
    macc, wacc, lacc = z, z, z
    for k in range(blk_rows // _STRIPE):
        p = p_ref[k * _STRIPE:(k + 1) * _STRIPE, :]
        l = l_ref[k * _STRIPE:(k + 1) * _STRIPE, :]
        u = jnp.exp(-jnp.abs(p))
        wacc = wacc + jnp.log(u + 1.0)
        macc = macc + (jnp.maximum(p, 0.0) - p * l)
        lacc = lacc + l
    macc_ref[...] += macc
    wacc_ref[...] += wacc
    lacc_ref[...] += lacc

    @pl.when(i == nblk - 1)
    def _fin():
        num_pos = jnp.sum(lacc_ref[...])
        least = float(int(total * LEAST_NEG_PERCENT))
        rand_neg = jnp.maximum(num_pos * float(RAND_NEG_RATIO), least)
        num_sampled_neg = jnp.minimum(rand_neg, float(total) - num_pos)
        balanced = num_pos + num_sampled_neg
        s = jnp.sum(macc_ref[...]) + jnp.sum(wacc_ref[...])
        out_ref[0] = LOSS_WEIGHT * s / balanced


def kernel(pred, label):
    total = pred.size
    rows = total // _LANES
    blk_rows = rows // _NBLK
    p2 = pred.reshape(rows, _LANES)
    l2 = label.reshape(rows, _LANES)
    out = pl.pallas_call(
        functools.partial(_body, nblk=_NBLK, total=total),
        grid=(_NBLK,),
        in_specs=[
            pl.BlockSpec((blk_rows, _LANES), lambda i: (i, 0)),
            pl.BlockSpec((blk_rows, _LANES), lambda i: (i, 0)),
        ],
        out_specs=pl.BlockSpec(memory_space=pltpu.SMEM),
        out_shape=jax.ShapeDtypeStruct((1,), jnp.float32),
        scratch_shapes=[
            pltpu.VMEM((_STRIPE, _LANES), jnp.float32),
            pltpu.VMEM((_STRIPE, _LANES), jnp.float32),
            pltpu.VMEM((_STRIPE, _LANES), jnp.float32),
        ],
    )(p2, l2)
    return out[0]


# speedup vs baseline: 7.2122x; 1.0384x over previous
import functools

import jax
import jax.numpy as jnp
from jax.experimental import pallas as pl
from jax.experimental.pallas import tpu as pltpu

RAND_NEG_RATIO = 3
LEAST_NEG_PERCENT = 0.05
LOSS_WEIGHT = 1.0

_LANES = 512
_NBLK = 4
_STRIPE = 8


def _body(p_ref, l_ref, out_ref, macc_ref, wacc_ref, lacc_ref, *, nblk, total):
    i = pl.program_id(0)

    @pl.when(i == 0)
    def _init():
        macc_ref[...] = jnp.zeros_like(macc_ref)
        wacc_ref[...] = jnp.zeros_like(wacc_ref)
        lacc_ref[...] = jnp.zeros_like(lacc_ref)

    blk_rows = p_ref.shape[0]
    z = jnp.zeros((_STRIPE, _LANES), jnp.float32)
    macc, wacc, lacc = z, z, z
    for k in range(blk_rows // _STRIPE):
        p = p_ref[k * _STRIPE:(k + 1) * _STRIPE, :]
        l = l_ref[k * _STRIPE:(k + 1) * _STRIPE, :]
        u = jnp.exp(-jnp.abs(p))
        wacc = wacc + jnp.log(u + 1.0)
        macc = macc + (jnp.maximum(p, 0.0) - p * l)
        lacc = lacc + l
    macc_ref[...] += macc
    wacc_ref[...] += wacc
    lacc_ref[...] += lacc

    @pl.when(i == nblk - 1)
    def _fin():
        num_pos = jnp.sum(lacc_ref[...])
        least = float(int(total * LEAST_NEG_PERCENT))
        rand_neg = jnp.maximum(num_pos * float(RAND_NEG_RATIO), least)
        num_sampled_neg = jnp.minimum(rand_neg, float(total) - num_pos)
        balanced = num_pos + num_sampled_neg
        s = jnp.sum(macc_ref[...]) + jnp.sum(wacc_ref[...])
        out_ref[0] = LOSS_WEIGHT * s / balanced


def kernel(pred, label):
    total = pred.size
    rows = total // _LANES
    blk_rows = rows // _NBLK
    p2 = pred.reshape(rows, _LANES)
    l2 = label.reshape(rows, _LANES)
    out = pl.pallas_call(
        functools.partial(_body, nblk=_NBLK, total=total),
        grid=(_NBLK,),
        in_specs=[
            pl.BlockSpec((blk_rows, _LANES), lambda i: (i, 0)),
            pl.BlockSpec((blk_rows, _LANES), lambda i: (i, 0)),
        ],
        out_specs=pl.BlockSpec(memory_space=pltpu.SMEM),
        out_shape=jax.ShapeDtypeStruct((1,), jnp.float32),
        scratch_shapes=[
            pltpu.VMEM((_STRIPE, _LANES), jnp.float32),
            pltpu.VMEM((_STRIPE, _LANES), jnp.float32),
            pltpu.VMEM((_STRIPE, _LANES), jnp.float32),
        ],
    )(p2, l2)
    return out[0]
